# Initial kernel scaffold; baseline (speedup 1.0000x reference)
import functools

import jax
import jax.numpy as jnp
from jax import lax
from jax.experimental import pallas as pl
from jax.experimental.pallas import tpu as pltpu
from jax.experimental.pallas import tpu_sc as plsc

N = 10000
E = 320000
IN_F = 128
OUT_F = 128

NC = 2
NS = 16
NW = NC * NS
LANES = 16

BLK = 128
E_TOT = E + N
NB = -(-E_TOT // (NW * BLK))
EP = NW * NB * BLK
DEN_W = 16
FE = OUT_F + DEN_W
N_PAD = 10240
RPW = N_PAD // NS


def _proj_body(x_ref, w_ref, asv_ref, adv_ref, h_ref, as_ref, ad_ref, m_ref):
    h = jnp.dot(x_ref[...], w_ref[...], preferred_element_type=jnp.float32)
    h_ref[...] = h
    a_s = jnp.dot(h, asv_ref[...], preferred_element_type=jnp.float32)
    a_d = jnp.dot(h, adv_ref[...], preferred_element_type=jnp.float32)
    as_ref[...] = a_s
    ad_ref[...] = a_d
    m = jnp.max(a_s) + jnp.max(a_d)
    m_ref[0, 0] = jnp.where(m > 0.0, m, 0.2 * m)


def _project(x, W, asv, adv):
    return pl.pallas_call(
        _proj_body,
        out_shape=(
            jax.ShapeDtypeStruct((N, OUT_F), jnp.float32),
            jax.ShapeDtypeStruct((N, 1), jnp.float32),
            jax.ShapeDtypeStruct((N, 1), jnp.float32),
            jax.ShapeDtypeStruct((1, 1), jnp.float32),
        ),
        out_specs=(
            pl.BlockSpec((N, OUT_F), lambda: (0, 0)),
            pl.BlockSpec((N, 1), lambda: (0, 0)),
            pl.BlockSpec((N, 1), lambda: (0, 0)),
            pl.BlockSpec(memory_space=pltpu.SMEM),
        ),
    )(x, W, asv, adv)


def _sc_body(src_hbm, dst_hbm, as_hbm, ad_hbm, m_hbm, hext_hbm, out_hbm,
             as_v, ad_v, src_v, dst_v, rows_v, att_v, m_v, acc_sh, sem):
    c = lax.axis_index("c")
    s = lax.axis_index("s")
    wid = s * NC + c

    pltpu.sync_copy(src_hbm.at[wid], src_v)
    pltpu.sync_copy(dst_hbm.at[wid], dst_v)
    pltpu.sync_copy(as_hbm, as_v)
    pltpu.sync_copy(ad_hbm, ad_v)
    pltpu.sync_copy(m_hbm, m_v)

    zeros16 = jnp.zeros((LANES,), jnp.float32)

    @pl.loop(0, BLK)
    def _zero_rows(r):
        for k in range(FE // LANES):
            rows_v[r, pl.ds(k * LANES, LANES)] = zeros16

    @pl.loop(0, RPW // BLK)
    def _paint(i):
        pltpu.sync_copy(rows_v, acc_sh.at[pl.ds(s * RPW + i * BLK, BLK)])

    plsc.subcore_barrier()

    m_vec = m_v[...]

    @pl.loop(0, NB)
    def _block(b):
        pltpu.async_copy(hext_hbm.at[src_v.at[b]], rows_v, sem).wait()

        @pl.loop(0, BLK // LANES)
        def _att(j):
            sl = pl.ds(j * LANES, LANES)
            s16 = src_v[b, sl]
            d16 = dst_v[b, sl]
            al = plsc.load_gather(as_v, [s16]) + plsc.load_gather(ad_v, [d16])
            al = jnp.where(al > 0.0, al, al * 0.2)
            att_v[sl] = jnp.exp(al - m_vec)

        @pl.loop(0, BLK)
        def _scale(r):
            w = plsc.load_gather(att_v, [jnp.full((LANES,), r, jnp.int32)])
            for k in range(FE // LANES):
                cs = pl.ds(k * LANES, LANES)
                rows_v[r, cs] = rows_v[r, cs] * w

        pltpu.sync_copy(rows_v, acc_sh.at[dst_v.at[b]], add=True)

    plsc.subcore_barrier()
    pltpu.sync_copy(acc_sh.at[pl.ds(s * RPW, RPW)],
                    out_hbm.at[pl.ds(c * N_PAD + s * RPW, RPW)])


def _sc_gat(src3, dst3, as_pad, ad_pad, m16, hext):
    mesh = plsc.VectorSubcoreMesh(core_axis_name="c", subcore_axis_name="s")
    k = pl.kernel(
        _sc_body,
        mesh=mesh,
        out_type=jax.ShapeDtypeStruct((NC * N_PAD, FE), jnp.float32),
        scratch_types=[
            pltpu.VMEM((N_PAD,), jnp.float32),
            pltpu.VMEM((N_PAD,), jnp.float32),
            pltpu.VMEM((NB, BLK), jnp.int32),
            pltpu.VMEM((NB, BLK), jnp.int32),
            pltpu.VMEM((BLK, FE), jnp.float32),
            pltpu.VMEM((BLK,), jnp.float32),
            pltpu.VMEM((LANES,), jnp.float32),
            pltpu.VMEM_SHARED((N_PAD, FE), jnp.float32),
            pltpu.SemaphoreType.DMA,
        ],
    )
    return k(src3, dst3, as_pad, ad_pad, m16, hext)


def _fin_body(p0_ref, p1_ref, bias_ref, o_ref):
    a = p0_ref[...] + p1_ref[...]
    num = a[:, :OUT_F]
    den = a[:, OUT_F:OUT_F + 1]
    o_ref[...] = num / (den + 1e-16) + bias_ref[...]


def _finalize(p0, p1, bias2):
    blk = 1000
    return pl.pallas_call(
        _fin_body,
        grid=(N // blk,),
        in_specs=[
            pl.BlockSpec((blk, FE), lambda i: (i, 0)),
            pl.BlockSpec((blk, FE), lambda i: (i, 0)),
            pl.BlockSpec((1, OUT_F), lambda i: (0, 0)),
        ],
        out_specs=pl.BlockSpec((blk, OUT_F), lambda i: (i, 0)),
        out_shape=jax.ShapeDtypeStruct((N, OUT_F), jnp.float32),
    )(p0, p1, bias2)


def kernel(x, edge_index, W, att_src, att_dst, bias):
    asv = att_src.reshape(OUT_F, 1).astype(jnp.float32)
    adv = att_dst.reshape(OUT_F, 1).astype(jnp.float32)

    h, a_s, a_d, m = _project(x, W, asv, adv)

    hext = jnp.concatenate([h, jnp.ones((N, DEN_W), jnp.float32)], axis=1)
    hext = jnp.pad(hext, ((0, N_PAD - N), (0, 0)))
    as_pad = jnp.pad(a_s[:, 0], (0, N_PAD - N))
    ad_pad = jnp.pad(a_d[:, 0], (0, N_PAD - N))
    m16 = jnp.broadcast_to(m.reshape(1), (LANES,))

    loop_idx = jnp.arange(N, dtype=jnp.int32)
    pad_idx = jnp.full((EP - E_TOT,), N, jnp.int32)
    src = jnp.concatenate([edge_index[0], loop_idx, pad_idx])
    dst = jnp.concatenate([edge_index[1], loop_idx, pad_idx])

    acc = _sc_gat(src.reshape(NW, NB, BLK), dst.reshape(NW, NB, BLK),
                  as_pad, ad_pad, m16, hext)
    acc = acc.reshape(NC, N_PAD, FE)

    out = _finalize(acc[0], acc[1], bias.reshape(1, OUT_F))
    return out


# speedup vs baseline: 19.6503x; 19.6503x over previous
import dataclasses
import functools

import jax
import jax.numpy as jnp
from jax import lax
from jax.experimental import pallas as pl
from jax.experimental.pallas import tpu as pltpu
from jax.experimental.pallas import tpu_sc as plsc

N = 10000
E = 320000
IN_F = 128
OUT_F = 128

NC = 2
NS = 16
NW = NC * NS
LANES = 16

BLK = 128
E_TOT = E + N
NB = -(-E_TOT // (NS * BLK))
EP = NS * NB * BLK
CW = 27
NCH = NB // CW
DEN_W = 16
HALF = OUT_F // 2
FEH = HALF + DEN_W
N_PAD = 10240
RPW = N_PAD // NS


def _proj_body(x_ref, w_ref, asv_ref, adv_ref, h_ref, as_ref, ad_ref, m_ref):
    h = jnp.dot(x_ref[...], w_ref[...], preferred_element_type=jnp.float32)
    h_ref[...] = h
    a_s = jnp.dot(h, asv_ref[...], preferred_element_type=jnp.float32)
    a_d = jnp.dot(h, adv_ref[...], preferred_element_type=jnp.float32)
    as_ref[...] = a_s
    ad_ref[...] = a_d
    m = jnp.max(a_s) + jnp.max(a_d)
    m_ref[0, 0] = jnp.where(m > 0.0, m, 0.2 * m)


def _project(x, W, asv, adv):
    return pl.pallas_call(
        _proj_body,
        out_shape=(
            jax.ShapeDtypeStruct((N, OUT_F), jnp.float32),
            jax.ShapeDtypeStruct((N, 1), jnp.float32),
            jax.ShapeDtypeStruct((N, 1), jnp.float32),
            jax.ShapeDtypeStruct((1, 1), jnp.float32),
        ),
        out_specs=(
            pl.BlockSpec((N, OUT_F), lambda: (0, 0)),
            pl.BlockSpec((N, 1), lambda: (0, 0)),
            pl.BlockSpec((N, 1), lambda: (0, 0)),
            pl.BlockSpec(memory_space=pltpu.SMEM),
        ),
    )(x, W, asv, adv)


def _sc_body(src_hbm, dst_hbm, as_hbm, ad_hbm, m_hbm, hext0_hbm, hext1_hbm,
             out_hbm, as_v, ad_v, src_v, dst_v, rows_v, att_v, m_v, acc_sh,
             sem):
    c = lax.axis_index("c")
    s = lax.axis_index("s")

    pltpu.sync_copy(as_hbm, as_v)
    pltpu.sync_copy(ad_hbm, ad_v)
    pltpu.sync_copy(m_hbm, m_v)

    zeros16 = jnp.zeros((LANES,), jnp.float32)

    @pl.loop(0, BLK)
    def _zero_rows(r):
        for k in range(FEH // LANES):
            rows_v[r, pl.ds(k * LANES, LANES)] = zeros16

    @pl.loop(0, RPW // BLK)
    def _paint(i):
        pltpu.sync_copy(rows_v, acc_sh.at[pl.ds(s * RPW + i * BLK, BLK)])

    plsc.subcore_barrier()

    m_vec = m_v[...]

    def _run(hext_hbm):
        @pl.loop(0, NCH)
        def _chunk(ch):
            pltpu.sync_copy(src_hbm.at[s * NCH + ch], src_v)
            pltpu.sync_copy(dst_hbm.at[s * NCH + ch], dst_v)

            @pl.loop(0, CW)
            def _block(b):
                pltpu.async_copy(hext_hbm.at[src_v.at[b]], rows_v, sem).wait()

                @pl.loop(0, BLK // LANES)
                def _att(j):
                    sl = pl.ds(j * LANES, LANES)
                    s16 = src_v[b, sl]
                    d16 = dst_v[b, sl]
                    al = (plsc.load_gather(as_v, [s16])
                          + plsc.load_gather(ad_v, [d16]))
                    al = jnp.where(al > 0.0, al, al * 0.2)
                    att_v[sl] = jnp.exp(al - m_vec)

                @pl.loop(0, BLK)
                def _scale(r):
                    w = plsc.load_gather(
                        att_v, [jnp.full((LANES,), r, jnp.int32)])
                    for k in range(FEH // LANES):
                        cs = pl.ds(k * LANES, LANES)
                        rows_v[r, cs] = rows_v[r, cs] * w

                pltpu.sync_copy(rows_v, acc_sh.at[dst_v.at[b]], add=True)

    @pl.when(c == 0)
    def _core0():
        _run(hext0_hbm)

    @pl.when(c == 1)
    def _core1():
        _run(hext1_hbm)

    plsc.subcore_barrier()
    pltpu.sync_copy(acc_sh.at[pl.ds(s * RPW, RPW)],
                    out_hbm.at[pl.ds(c * N_PAD + s * RPW, RPW)])


def _sc_gat(src4, dst4, as_pad, ad_pad, m16, hext0, hext1):
    mesh = plsc.VectorSubcoreMesh(core_axis_name="c", subcore_axis_name="s")
    cp = pltpu.CompilerParams()
    if "needs_layout_passes" in pltpu.CompilerParams.__dataclass_fields__:
        cp = dataclasses.replace(cp, needs_layout_passes=False)
    if "use_tc_tiling_on_sc" in pltpu.CompilerParams.__dataclass_fields__:
        cp = dataclasses.replace(cp, use_tc_tiling_on_sc=False)
    k = pl.kernel(
        _sc_body,
        mesh=mesh,
        compiler_params=cp,
        out_type=jax.ShapeDtypeStruct((NC * N_PAD, FEH), jnp.float32),
        scratch_types=[
            pltpu.VMEM((N_PAD,), jnp.float32),
            pltpu.VMEM((N_PAD,), jnp.float32),
            pltpu.VMEM((CW, BLK), jnp.int32),
            pltpu.VMEM((CW, BLK), jnp.int32),
            pltpu.VMEM((BLK, FEH), jnp.float32),
            pltpu.VMEM((BLK,), jnp.float32),
            pltpu.VMEM((LANES,), jnp.float32),
            pltpu.VMEM_SHARED((N_PAD, FEH), jnp.float32),
            pltpu.SemaphoreType.DMA,
        ],
    )
    return k(src4, dst4, as_pad, ad_pad, m16, hext0, hext1)


def _fin_body(p0_ref, p1_ref, bias_ref, o_ref):
    p0 = p0_ref[...]
    p1 = p1_ref[...]
    b = bias_ref[...]
    o_ref[:, :HALF] = (p0[:, :HALF] / (p0[:, HALF:HALF + 1] + 1e-16)
                       + b[:, :HALF])
    o_ref[:, HALF:] = (p1[:, :HALF] / (p1[:, HALF:HALF + 1] + 1e-16)
                       + b[:, HALF:])


def _finalize(p0, p1, bias2):
    blk = 1000
    return pl.pallas_call(
        _fin_body,
        grid=(N // blk,),
        in_specs=[
            pl.BlockSpec((blk, FEH), lambda i: (i, 0)),
            pl.BlockSpec((blk, FEH), lambda i: (i, 0)),
            pl.BlockSpec((1, OUT_F), lambda i: (0, 0)),
        ],
        out_specs=pl.BlockSpec((blk, OUT_F), lambda i: (i, 0)),
        out_shape=jax.ShapeDtypeStruct((N, OUT_F), jnp.float32),
    )(p0, p1, bias2)


def kernel(x, edge_index, W, att_src, att_dst, bias):
    asv = att_src.reshape(OUT_F, 1).astype(jnp.float32)
    adv = att_dst.reshape(OUT_F, 1).astype(jnp.float32)

    h, a_s, a_d, m = _project(x, W, asv, adv)

    ones = jnp.ones((N, DEN_W), jnp.float32)
    hext0 = jnp.pad(jnp.concatenate([h[:, :HALF], ones], axis=1),
                    ((0, N_PAD - N), (0, 0)))
    hext1 = jnp.pad(jnp.concatenate([h[:, HALF:], ones], axis=1),
                    ((0, N_PAD - N), (0, 0)))
    as_pad = jnp.pad(a_s[:, 0], (0, N_PAD - N))
    ad_pad = jnp.pad(a_d[:, 0], (0, N_PAD - N))
    m16 = jnp.broadcast_to(m.reshape(1), (LANES,))

    loop_idx = jnp.arange(N, dtype=jnp.int32)
    pad_idx = jnp.full((EP - E_TOT,), N, jnp.int32)
    src = jnp.concatenate([edge_index[0], loop_idx, pad_idx])
    dst = jnp.concatenate([edge_index[1], loop_idx, pad_idx])

    acc = _sc_gat(src.reshape(NS * NCH, CW, BLK),
                  dst.reshape(NS * NCH, CW, BLK),
                  as_pad, ad_pad, m16, hext0, hext1)
    acc = acc.reshape(NC, N_PAD, FEH)

    out = _finalize(acc[0], acc[1], bias.reshape(1, OUT_F))
    return out
